# baseline (speedup 1.0000x reference)
import functools

import jax
import jax.numpy as jnp
from jax import lax
from jax.experimental import pallas as pl
from jax.experimental.pallas import tpu as pltpu
from jax.experimental.pallas import tpu_sc as plsc

VOCAB = 1000000
CONTENT_DIM = 64
TIME_DIM = 16
OUT_DIM = CONTENT_DIM + TIME_DIM
BATCH = 4096
SEQ = 200

_INFO = plsc.get_sparse_core_info()
NC, NS = _INFO.num_cores, _INFO.num_subcores
NW = NC * NS

CHUNK = 128
BT = BATCH // CHUNK
DT = OUT_DIM // 8
N_CHUNKS = SEQ * BT
CH_PER_W = N_CHUNKS // NW


def _gather_body(ids_hbm, content_hbm, time_hbm, out_hbm,
                 idx0, idx1, rc0, rc1, rt0, rt1, qb0, qb1,
                 sem_c0, sem_c1, sem_t0, sem_t1, sem_o):
    wid = lax.axis_index("s") * NC + lax.axis_index("c")
    base = wid * CH_PER_W
    end = base + CH_PER_W
    lane = lax.iota(jnp.int32, 16)
    jidx = [k * 16 + lane for k in range(CHUNK // 16)]

    idx_v = [idx0, idx1]
    rows_c = [rc0, rc1]
    rows_t = [rt0, rt1]
    qbuf = [qb0, qb1]
    sem_c = [sem_c0, sem_c1]
    sem_t = [sem_t0, sem_t1]

    def gathers(slot):
        return (
            pltpu.make_async_copy(
                content_hbm.at[idx_v[slot]], rows_c[slot], sem_c[slot]),
            pltpu.make_async_copy(
                time_hbm.at[idx_v[slot]], rows_t[slot], sem_t[slot]),
        )

    def issue(c, slot):
        s = c // BT
        bt = c % BT
        pltpu.sync_copy(
            ids_hbm.at[pl.ds(s * BATCH + bt * CHUNK, CHUNK)], idx_v[slot])
        g_c, g_t = gathers(slot)
        g_c.start()
        g_t.start()

    issue(base, 0)

    def pair_body(i, carry):
        c0 = base + 2 * i
        for b in range(2):
            c = c0 + b
            slot, other = b, 1 - b

            @pl.when(c + 1 < end)
            def _prefetch():
                issue(c + 1, other)

            g_c, g_t = gathers(slot)
            g_c.wait()
            g_t.wait()

            for d in range(OUT_DIM):
                if d < CONTENT_DIM:
                    src, dd = rows_c[slot], d
                else:
                    src, dd = rows_t[slot], d - CONTENT_DIM
                dsplat = jnp.full((16,), dd, jnp.int32)
                for k in range(CHUNK // 16):
                    x = plsc.load_gather(src, [jidx[k], dsplat])
                    qbuf[slot][d, pl.ds(k * 16, 16)] = x

            s = c // BT
            bt = c % BT
            outs = [
                pltpu.make_async_copy(
                    qbuf[slot].at[pl.ds(dt * 8, 8), :],
                    out_hbm.at[s, dt, bt], sem_o)
                for dt in range(DT)
            ]
            for o in outs:
                o.start()
            for o in outs:
                o.wait()
        return carry

    lax.fori_loop(0, CH_PER_W // 2, pair_body, 0)


@jax.jit
def kernel(token_ids, content_table, time_table):
    ids = token_ids.T.reshape(BATCH * SEQ)

    mesh = plsc.VectorSubcoreMesh(core_axis_name="c", subcore_axis_name="s")
    q = pl.kernel(
        _gather_body,
        out_type=jax.ShapeDtypeStruct((SEQ, DT, BT, 8, CHUNK), jnp.float32),
        mesh=mesh,
        scratch_types=[
            pltpu.VMEM((CHUNK,), jnp.int32),
            pltpu.VMEM((CHUNK,), jnp.int32),
            pltpu.VMEM((CHUNK, CONTENT_DIM), jnp.float32),
            pltpu.VMEM((CHUNK, CONTENT_DIM), jnp.float32),
            pltpu.VMEM((CHUNK, TIME_DIM), jnp.float32),
            pltpu.VMEM((CHUNK, TIME_DIM), jnp.float32),
            pltpu.VMEM((OUT_DIM, CHUNK), jnp.float32),
            pltpu.VMEM((OUT_DIM, CHUNK), jnp.float32),
            pltpu.SemaphoreType.DMA,
            pltpu.SemaphoreType.DMA,
            pltpu.SemaphoreType.DMA,
            pltpu.SemaphoreType.DMA,
            pltpu.SemaphoreType.DMA,
        ],
        compiler_params=pltpu.CompilerParams(
            use_tc_tiling_on_sc=False, needs_layout_passes=False),
    )(ids, content_table, time_table)
    return q.transpose(2, 4, 0, 1, 3).reshape(BATCH, SEQ, OUT_DIM)


# speedup vs baseline: 1.7085x; 1.7085x over previous
import functools

import jax
import jax.numpy as jnp
from jax import lax
from jax.experimental import pallas as pl
from jax.experimental.pallas import tpu as pltpu
from jax.experimental.pallas import tpu_sc as plsc

VOCAB = 1000000
CONTENT_DIM = 64
TIME_DIM = 16
OUT_DIM = CONTENT_DIM + TIME_DIM
BATCH = 4096
SEQ = 200

_INFO = plsc.get_sparse_core_info()
NC, NS = _INFO.num_cores, _INFO.num_subcores
NW = NC * NS

CHUNK = 128
BT = BATCH // CHUNK
DT = OUT_DIM // 8
N_CHUNKS = SEQ * BT
CH_PER_W = N_CHUNKS // NW


QPAD = CHUNK + 1


def _gather_body(ids_hbm, content_hbm, time_hbm, out_hbm,
                 idx0, idx1, rc0, rc1, rt0, rt1, qb0, qb1,
                 sem_i0, sem_i1, sem_c0, sem_c1, sem_t0, sem_t1, sem_o):
    wid = lax.axis_index("s") * NC + lax.axis_index("c")
    base = wid * CH_PER_W
    end = base + CH_PER_W
    lane = lax.iota(jnp.int32, 16)
    didx_c = [k * 16 + lane for k in range(CONTENT_DIM // 16)]
    didx_t = CONTENT_DIM + lane

    idx_v = [idx0, idx1]
    rows_c = [rc0, rc1]
    rows_t = [rt0, rt1]
    qbuf = [qb0, qb1]
    sem_i = [sem_i0, sem_i1]
    sem_c = [sem_c0, sem_c1]
    sem_t = [sem_t0, sem_t1]

    def idx_copy(c, slot):
        s = c // BT
        bt = c % BT
        return pltpu.make_async_copy(
            ids_hbm.at[pl.ds(s * BATCH + bt * CHUNK, CHUNK)],
            idx_v[slot], sem_i[slot])

    def gathers(slot):
        return (
            pltpu.make_async_copy(
                content_hbm.at[idx_v[slot]], rows_c[slot], sem_c[slot]),
            pltpu.make_async_copy(
                time_hbm.at[idx_v[slot]], rows_t[slot], sem_t[slot]),
        )

    idx_copy(base, 0).start()
    idx_copy(base + 1, 1).start()
    idx_copy(base, 0).wait()
    g0_c, g0_t = gathers(0)
    g0_c.start()
    g0_t.start()

    def pair_body(i, carry):
        c0 = base + 2 * i
        for b in range(2):
            c = c0 + b
            slot, other = b, 1 - b

            g_c, g_t = gathers(slot)
            g_c.wait()
            g_t.wait()

            @pl.when(c + 2 < end)
            def _idx_pf():
                idx_copy(c + 2, slot).start()

            @pl.when(c + 1 < end)
            def _gather_pf():
                idx_copy(c + 1, other).wait()
                n_c, n_t = gathers(other)
                n_c.start()
                n_t.start()

            for j in range(CHUNK):
                jv = jnp.full((16,), j, jnp.int32)
                for k in range(CONTENT_DIM // 16):
                    x = rows_c[slot][j, pl.ds(k * 16, 16)]
                    plsc.store_scatter(qbuf[slot], [didx_c[k], jv], x)
                x = rows_t[slot][j, pl.ds(0, 16)]
                plsc.store_scatter(qbuf[slot], [didx_t, jv], x)

            s = c // BT
            bt = c % BT
            outs = [
                pltpu.make_async_copy(
                    qbuf[slot].at[pl.ds(dt * 8, 8), pl.ds(0, CHUNK)],
                    out_hbm.at[s, dt, bt], sem_o)
                for dt in range(DT)
            ]
            for o in outs:
                o.start()
            for o in outs:
                o.wait()
        return carry

    lax.fori_loop(0, CH_PER_W // 2, pair_body, 0)


@jax.jit
def kernel(token_ids, content_table, time_table):
    ids = token_ids.T.reshape(BATCH * SEQ)

    mesh = plsc.VectorSubcoreMesh(core_axis_name="c", subcore_axis_name="s")
    q = pl.kernel(
        _gather_body,
        out_type=jax.ShapeDtypeStruct((SEQ, DT, BT, 8, CHUNK), jnp.float32),
        mesh=mesh,
        scratch_types=[
            pltpu.VMEM((CHUNK,), jnp.int32),
            pltpu.VMEM((CHUNK,), jnp.int32),
            pltpu.VMEM((CHUNK, CONTENT_DIM), jnp.float32),
            pltpu.VMEM((CHUNK, CONTENT_DIM), jnp.float32),
            pltpu.VMEM((CHUNK, TIME_DIM), jnp.float32),
            pltpu.VMEM((CHUNK, TIME_DIM), jnp.float32),
            pltpu.VMEM((OUT_DIM, QPAD), jnp.float32),
            pltpu.VMEM((OUT_DIM, QPAD), jnp.float32),
            pltpu.SemaphoreType.DMA,
            pltpu.SemaphoreType.DMA,
            pltpu.SemaphoreType.DMA,
            pltpu.SemaphoreType.DMA,
            pltpu.SemaphoreType.DMA,
            pltpu.SemaphoreType.DMA,
            pltpu.SemaphoreType.DMA,
        ],
        compiler_params=pltpu.CompilerParams(
            use_tc_tiling_on_sc=False, needs_layout_passes=False),
    )(ids, content_table, time_table)
    return q.transpose(2, 4, 0, 1, 3).reshape(BATCH, SEQ, OUT_DIM)


# speedup vs baseline: 2.2696x; 1.3284x over previous
import functools

import jax
import jax.numpy as jnp
from jax import lax
from jax.experimental import pallas as pl
from jax.experimental.pallas import tpu as pltpu
from jax.experimental.pallas import tpu_sc as plsc

VOCAB = 1000000
CONTENT_DIM = 64
TIME_DIM = 16
OUT_DIM = CONTENT_DIM + TIME_DIM
BATCH = 4096
SEQ = 200

_INFO = plsc.get_sparse_core_info()
NC, NS = _INFO.num_cores, _INFO.num_subcores
NW = NC * NS

CHUNK = 256
BT = BATCH // CHUNK
DT = OUT_DIM // 8
N_CHUNKS = SEQ * BT
CH_PER_W = N_CHUNKS // NW


QPAD = 129


NT_FULL = VOCAB // 128


def _relayout_body(ct_hbm, tt_hbm, tailc_hbm, tailt_hbm, crm_hbm, trm_hbm,
                   cin0, cin1, cout0, cout1, tin0, tin1, tout0, tout1,
                   sem_ci0, sem_ci1, sem_co0, sem_co1,
                   sem_ti0, sem_ti1, sem_to0, sem_to1):
    wid = lax.axis_index("s") * NC + lax.axis_index("c")
    lane = lax.iota(jnp.int32, 16)

    cin = [cin0, cin1]
    cout = [cout0, cout1]
    tin = [tin0, tin1]
    tout = [tout0, tout1]
    sem_ci = [sem_ci0, sem_ci1]
    sem_co = [sem_co0, sem_co1]
    sem_ti = [sem_ti0, sem_ti1]
    sem_to = [sem_to0, sem_to1]

    NT_W = (NT_FULL + NW - 1) // NW

    def tile_of(u):
        return wid + NW * u

    def c_in_copies(tt, slot):
        return [
            pltpu.make_async_copy(
                ct_hbm.at[pl.ds(dt * 8, 8), pl.ds(tt * 128, 128)],
                cin[slot].at[pl.ds(dt * 8, 8), :], sem_ci[slot])
            for dt in range(CONTENT_DIM // 8)
        ]

    def c_out_copy(tt, slot):
        return pltpu.make_async_copy(
            cout[slot], crm_hbm.at[pl.ds(tt * 64, 64), :], sem_co[slot])

    for cp in c_in_copies(tile_of(0), 0):
        cp.start()

    jvec_c = [16 * jb + lane for jb in range(8)]
    pvec_c = [8 * jb + lane // 2 for jb in range(8)]
    ovec_t = [(16 * jb + lane) // 8 for jb in range(8)]
    qbase = (lane % 2) * CONTENT_DIM

    def c_transpose(slot):
        def r_body(r, carry):
            dv0 = (lane + r) & 15
            for kb in range(CONTENT_DIM // 16):
                dv = dv0 + 16 * kb
                qv = qbase + 16 * kb + dv0
                for jb in range(8):
                    x = plsc.load_gather(cin[slot], [dv, jvec_c[jb]])
                    plsc.store_scatter(cout[slot], [pvec_c[jb], qv], x)
            return carry

        lax.fori_loop(0, 16, r_body, 0)

    def c_pair(i, carry):
        for b in range(2):
            u = 2 * i + b
            slot, other = b, 1 - b
            tt = tile_of(u)

            @pl.when(tt < NT_FULL)
            def _step():
                for cp in c_in_copies(tt, slot):
                    cp.wait()

                @pl.when(tile_of(u + 1) < NT_FULL)
                def _pf():
                    for cp in c_in_copies(tile_of(u + 1), other):
                        cp.start()

                @pl.when(u >= 2)
                def _drain():
                    c_out_copy(tile_of(u - 2), slot).wait()

                c_transpose(slot)
                c_out_copy(tt, slot).start()
        return carry

    lax.fori_loop(0, (NT_W + 2) // 2, c_pair, 0)
    c_out_copy(tile_of(0), 0).wait()
    c_out_copy(tile_of(0), 1).wait()

    def t_in_copies(tt, slot):
        return [
            pltpu.make_async_copy(
                tt_hbm.at[pl.ds(dt * 8, 8), pl.ds(tt * 128, 128)],
                tin[slot].at[pl.ds(dt * 8, 8), :], sem_ti[slot])
            for dt in range(TIME_DIM // 8)
        ]

    def t_out_copy(tt, slot):
        return pltpu.make_async_copy(
            tout[slot], trm_hbm.at[pl.ds(tt * 16, 16), :], sem_to[slot])

    for cp in t_in_copies(tile_of(0), 0):
        cp.start()

    def t_transpose(slot):
        def r_body(r, carry):
            dv0 = (lane + r) & 15
            qv = (lane & 7) * 16 + dv0
            for jb in range(8):
                x = plsc.load_gather(tin[slot], [dv0, jvec_c[jb]])
                plsc.store_scatter(tout[slot], [ovec_t[jb], qv], x)
            return carry

        lax.fori_loop(0, 16, r_body, 0)

    def t_pair(i, carry):
        for b in range(2):
            u = 2 * i + b
            slot, other = b, 1 - b
            tt = tile_of(u)

            @pl.when(tt < NT_FULL)
            def _step():
                for cp in t_in_copies(tt, slot):
                    cp.wait()

                @pl.when(tile_of(u + 1) < NT_FULL)
                def _pf():
                    for cp in t_in_copies(tile_of(u + 1), other):
                        cp.start()

                @pl.when(u >= 2)
                def _drain():
                    t_out_copy(tile_of(u - 2), slot).wait()

                t_transpose(slot)
                t_out_copy(tt, slot).start()
        return carry

    lax.fori_loop(0, (NT_W + 2) // 2, t_pair, 0)
    t_out_copy(tile_of(0), 0).wait()
    t_out_copy(tile_of(0), 1).wait()

    @pl.when(wid == 0)
    def _tails():
        pltpu.sync_copy(tailc_hbm, cin[0].at[pl.ds(0, 32), :])
        pltpu.sync_copy(cin[0].at[pl.ds(0, 32), :],
                        crm_hbm.at[pl.ds(NT_FULL * 64, 32), :])
        pltpu.sync_copy(tailt_hbm, tin[0].at[pl.ds(0, 8), :])
        pltpu.sync_copy(tin[0].at[pl.ds(0, 8), :],
                        trm_hbm.at[pl.ds(NT_FULL * 16, 8), :])


def _gather_body(ids_hbm, content_hbm, time_hbm, out_hbm,
                 idx0, idx1, rc0, rc1, rt0, rt1, qb0, qb1,
                 sem_i0, sem_i1, sem_c0, sem_c1, sem_t0, sem_t1, sem_o):
    wid = lax.axis_index("s") * NC + lax.axis_index("c")
    base = wid * CH_PER_W
    end = base + CH_PER_W
    lane = lax.iota(jnp.int32, 16)
    didx_c = [k * 16 + lane for k in range(CONTENT_DIM // 16)]
    didx_t = CONTENT_DIM + lane

    idx_v = [idx0, idx1]
    rows_c = [rc0, rc1]
    rows_t = [rt0, rt1]
    qbuf = [qb0, qb1]
    sem_i = [sem_i0, sem_i1]
    sem_c = [sem_c0, sem_c1]
    sem_t = [sem_t0, sem_t1]

    def idx_copy(c, slot):
        return pltpu.make_async_copy(
            ids_hbm.at[pl.ds(2 * c, 2), :], idx_v[slot], sem_i[slot])

    def gathers(slot):
        cps = []
        for h in range(2):
            cps.append(pltpu.make_async_copy(
                content_hbm.at[idx_v[slot].at[h]],
                rows_c[slot].at[pl.ds(h * 128, 128), :], sem_c[slot]))
            cps.append(pltpu.make_async_copy(
                time_hbm.at[idx_v[slot].at[h]],
                rows_t[slot].at[pl.ds(h * 128, 128), :], sem_t[slot]))
        return cps

    idx_copy(base, 0).start()
    idx_copy(base + 1, 1).start()
    idx_copy(base, 0).wait()
    for g in gathers(0):
        g.start()

    def pair_body(i, carry):
        c0 = base + 2 * i
        for b in range(2):
            c = c0 + b
            slot, other = b, 1 - b

            for g in gathers(slot):
                g.wait()

            @pl.when(c + 2 < end)
            def _idx_pf():
                idx_copy(c + 2, slot).start()

            @pl.when(c + 1 < end)
            def _gather_pf():
                idx_copy(c + 1, other).wait()
                for g in gathers(other):
                    g.start()

            def tr_body(j, carry2):
                jh = jnp.full((16,), j // 128, jnp.int32)
                jv = jnp.full((16,), j % 128, jnp.int32)
                for k in range(CONTENT_DIM // 16):
                    x = rows_c[slot][j, pl.ds(k * 16, 16)]
                    plsc.store_scatter(qbuf[slot], [jh, didx_c[k], jv], x)
                x = rows_t[slot][j, pl.ds(0, 16)]
                plsc.store_scatter(qbuf[slot], [jh, didx_t, jv], x)
                return carry2

            lax.fori_loop(0, CHUNK, tr_body, 0, unroll=8)

            s = c // BT
            b2 = c % BT
            outs = [
                pltpu.make_async_copy(
                    qbuf[slot].at[:, pl.ds(dt * 8, 8), pl.ds(0, 128)],
                    out_hbm.at[s, dt, pl.ds(2 * b2, 2)], sem_o)
                for dt in range(DT)
            ]
            for o in outs:
                o.start()
            for o in outs:
                o.wait()
        return carry

    lax.fori_loop(0, CH_PER_W // 2, pair_body, 0)


@jax.jit
def kernel(token_ids, content_table, time_table):
    ids = token_ids.T.reshape(BATCH * SEQ // 128, 128)
    mesh = plsc.VectorSubcoreMesh(core_axis_name="c", subcore_axis_name="s")

    tail0 = NT_FULL * 128
    crm, trm = pl.kernel(
        _relayout_body,
        out_type=(
            jax.ShapeDtypeStruct((VOCAB // 2, 128), jnp.float32),
            jax.ShapeDtypeStruct((VOCAB // 8, 128), jnp.float32),
        ),
        mesh=mesh,
        scratch_types=[
            pltpu.VMEM((CONTENT_DIM, 128), jnp.float32),
            pltpu.VMEM((CONTENT_DIM, 128), jnp.float32),
            pltpu.VMEM((CONTENT_DIM, 128), jnp.float32),
            pltpu.VMEM((CONTENT_DIM, 128), jnp.float32),
            pltpu.VMEM((TIME_DIM, 128), jnp.float32),
            pltpu.VMEM((TIME_DIM, 128), jnp.float32),
            pltpu.VMEM((TIME_DIM, 128), jnp.float32),
            pltpu.VMEM((TIME_DIM, 128), jnp.float32),
        ] + [pltpu.SemaphoreType.DMA] * 8,
        compiler_params=pltpu.CompilerParams(
            use_tc_tiling_on_sc=True, needs_layout_passes=False),
    )(
        content_table.T,
        time_table.T,
        content_table[tail0:].reshape(32, 128),
        time_table[tail0:].reshape(8, 128),
    )
    content_rm = crm.reshape(VOCAB, CONTENT_DIM)
    time_rm = trm.reshape(VOCAB, TIME_DIM)

    q = pl.kernel(
        _gather_body,
        out_type=jax.ShapeDtypeStruct((SEQ, DT, 2 * BT, 8, 128), jnp.float32),
        mesh=mesh,
        scratch_types=[
            pltpu.VMEM((2, 128), jnp.int32),
            pltpu.VMEM((2, 128), jnp.int32),
            pltpu.VMEM((CHUNK, CONTENT_DIM), jnp.float32),
            pltpu.VMEM((CHUNK, CONTENT_DIM), jnp.float32),
            pltpu.VMEM((CHUNK, TIME_DIM), jnp.float32),
            pltpu.VMEM((CHUNK, TIME_DIM), jnp.float32),
            pltpu.VMEM((2, OUT_DIM, QPAD), jnp.float32),
            pltpu.VMEM((2, OUT_DIM, QPAD), jnp.float32),
            pltpu.SemaphoreType.DMA,
            pltpu.SemaphoreType.DMA,
            pltpu.SemaphoreType.DMA,
            pltpu.SemaphoreType.DMA,
            pltpu.SemaphoreType.DMA,
            pltpu.SemaphoreType.DMA,
            pltpu.SemaphoreType.DMA,
        ],
        compiler_params=pltpu.CompilerParams(
            use_tc_tiling_on_sc=False, needs_layout_passes=False),
    )(ids, content_rm, time_rm)
    return q.transpose(2, 4, 0, 1, 3).reshape(BATCH, SEQ, OUT_DIM)


# speedup vs baseline: 2.3093x; 1.0175x over previous
import functools

import jax
import jax.numpy as jnp
from jax import lax
from jax.experimental import pallas as pl
from jax.experimental.pallas import tpu as pltpu
from jax.experimental.pallas import tpu_sc as plsc

VOCAB = 1000000
CONTENT_DIM = 64
TIME_DIM = 16
OUT_DIM = CONTENT_DIM + TIME_DIM
BATCH = 4096
SEQ = 200

_INFO = plsc.get_sparse_core_info()
NC, NS = _INFO.num_cores, _INFO.num_subcores
NW = NC * NS

CHUNK = 256
BT = BATCH // CHUNK
DT = OUT_DIM // 8
N_CHUNKS = SEQ * BT
CH_PER_W = N_CHUNKS // NW


QPAD = 129


NT_FULL = VOCAB // 128


def _relayout_body(ct_hbm, tt_hbm, tailc_hbm, tailt_hbm, crm_hbm, trm_hbm,
                   cin0, cin1, cout0, cout1, tin0, tin1, tout0, tout1,
                   sem_ci0, sem_ci1, sem_co0, sem_co1,
                   sem_ti0, sem_ti1, sem_to0, sem_to1):
    wid = lax.axis_index("s") * NC + lax.axis_index("c")
    lane = lax.iota(jnp.int32, 16)

    cin = [cin0, cin1]
    cout = [cout0, cout1]
    tin = [tin0, tin1]
    tout = [tout0, tout1]
    sem_ci = [sem_ci0, sem_ci1]
    sem_co = [sem_co0, sem_co1]
    sem_ti = [sem_ti0, sem_ti1]
    sem_to = [sem_to0, sem_to1]

    NT_W = (NT_FULL + NW - 1) // NW

    def tile_of(u):
        return wid + NW * u

    def c_in_copies(tt, slot):
        return [
            pltpu.make_async_copy(
                ct_hbm.at[pl.ds(dt * 8, 8), pl.ds(tt * 128, 128)],
                cin[slot].at[pl.ds(dt * 8, 8), :], sem_ci[slot])
            for dt in range(CONTENT_DIM // 8)
        ]

    def c_out_copy(tt, slot):
        return pltpu.make_async_copy(
            cout[slot], crm_hbm.at[pl.ds(tt * 64, 64), :], sem_co[slot])

    for cp in c_in_copies(tile_of(0), 0):
        cp.start()

    jvec_c = [16 * jb + lane for jb in range(8)]
    pvec_c = [8 * jb + lane // 2 for jb in range(8)]
    ovec_t = [(16 * jb + lane) // 8 for jb in range(8)]
    qbase = (lane % 2) * CONTENT_DIM

    def c_transpose(slot):
        def r_body(r, carry):
            dv0 = (lane + r) & 15
            for kb in range(CONTENT_DIM // 16):
                dv = dv0 + 16 * kb
                qv = qbase + 16 * kb + dv0
                for jb in range(8):
                    x = plsc.load_gather(cin[slot], [dv, jvec_c[jb]])
                    plsc.store_scatter(cout[slot], [pvec_c[jb], qv], x)
            return carry

        lax.fori_loop(0, 16, r_body, 0)

    def c_pair(i, carry):
        for b in range(2):
            u = 2 * i + b
            slot, other = b, 1 - b
            tt = tile_of(u)

            @pl.when(tt < NT_FULL)
            def _step():
                for cp in c_in_copies(tt, slot):
                    cp.wait()

                @pl.when(tile_of(u + 1) < NT_FULL)
                def _pf():
                    for cp in c_in_copies(tile_of(u + 1), other):
                        cp.start()

                @pl.when(u >= 2)
                def _drain():
                    c_out_copy(tile_of(u - 2), slot).wait()

                c_transpose(slot)
                c_out_copy(tt, slot).start()
        return carry

    lax.fori_loop(0, (NT_W + 2) // 2, c_pair, 0)
    c_out_copy(tile_of(0), 0).wait()
    c_out_copy(tile_of(0), 1).wait()

    def t_in_copies(tt, slot):
        return [
            pltpu.make_async_copy(
                tt_hbm.at[pl.ds(dt * 8, 8), pl.ds(tt * 128, 128)],
                tin[slot].at[pl.ds(dt * 8, 8), :], sem_ti[slot])
            for dt in range(TIME_DIM // 8)
        ]

    def t_out_copy(tt, slot):
        return pltpu.make_async_copy(
            tout[slot], trm_hbm.at[pl.ds(tt * 16, 16), :], sem_to[slot])

    for cp in t_in_copies(tile_of(0), 0):
        cp.start()

    def t_transpose(slot):
        def r_body(r, carry):
            dv0 = (lane + r) & 15
            qv = (lane & 7) * 16 + dv0
            for jb in range(8):
                x = plsc.load_gather(tin[slot], [dv0, jvec_c[jb]])
                plsc.store_scatter(tout[slot], [ovec_t[jb], qv], x)
            return carry

        lax.fori_loop(0, 16, r_body, 0)

    def t_pair(i, carry):
        for b in range(2):
            u = 2 * i + b
            slot, other = b, 1 - b
            tt = tile_of(u)

            @pl.when(tt < NT_FULL)
            def _step():
                for cp in t_in_copies(tt, slot):
                    cp.wait()

                @pl.when(tile_of(u + 1) < NT_FULL)
                def _pf():
                    for cp in t_in_copies(tile_of(u + 1), other):
                        cp.start()

                @pl.when(u >= 2)
                def _drain():
                    t_out_copy(tile_of(u - 2), slot).wait()

                t_transpose(slot)
                t_out_copy(tt, slot).start()
        return carry

    lax.fori_loop(0, (NT_W + 2) // 2, t_pair, 0)
    t_out_copy(tile_of(0), 0).wait()
    t_out_copy(tile_of(0), 1).wait()

    @pl.when(wid == 0)
    def _tails():
        pltpu.sync_copy(tailc_hbm, cin[0].at[pl.ds(0, 32), :])
        pltpu.sync_copy(cin[0].at[pl.ds(0, 32), :],
                        crm_hbm.at[pl.ds(NT_FULL * 64, 32), :])
        pltpu.sync_copy(tailt_hbm, tin[0].at[pl.ds(0, 8), :])
        pltpu.sync_copy(tin[0].at[pl.ds(0, 8), :],
                        trm_hbm.at[pl.ds(NT_FULL * 16, 8), :])


def _gather_body(ids_hbm, content_hbm, time_hbm, out_hbm,
                 idx0, idx1, rc0, rc1, rt0, rt1, qb0, qb1,
                 sem_i0, sem_i1, sem_c0, sem_c1, sem_t0, sem_t1,
                 sem_o0, sem_o1):
    wid = lax.axis_index("s") * NC + lax.axis_index("c")
    base = wid * CH_PER_W
    end = base + CH_PER_W
    lane = lax.iota(jnp.int32, 16)
    didx_c = [k * 16 + lane for k in range(CONTENT_DIM // 16)]
    didx_t = CONTENT_DIM + lane

    idx_v = [idx0, idx1]
    rows_c = [rc0, rc1]
    rows_t = [rt0, rt1]
    qbuf = [qb0, qb1]
    sem_i = [sem_i0, sem_i1]
    sem_c = [sem_c0, sem_c1]
    sem_t = [sem_t0, sem_t1]
    sem_o = [sem_o0, sem_o1]

    def out_copies(c, slot):
        s = c // BT
        b2 = c % BT
        return [
            pltpu.make_async_copy(
                qbuf[slot].at[:, pl.ds(dt * 8, 8), pl.ds(0, 128)],
                out_hbm.at[s, dt, pl.ds(2 * b2, 2)], sem_o[slot])
            for dt in range(DT)
        ]

    def idx_copy(c, slot):
        return pltpu.make_async_copy(
            ids_hbm.at[pl.ds(2 * c, 2), :], idx_v[slot], sem_i[slot])

    def gathers(slot):
        cps = []
        for h in range(2):
            cps.append(pltpu.make_async_copy(
                content_hbm.at[idx_v[slot].at[h]],
                rows_c[slot].at[pl.ds(h * 128, 128), :], sem_c[slot]))
            cps.append(pltpu.make_async_copy(
                time_hbm.at[idx_v[slot].at[h]],
                rows_t[slot].at[pl.ds(h * 128, 128), :], sem_t[slot]))
        return cps

    idx_copy(base, 0).start()
    idx_copy(base + 1, 1).start()
    idx_copy(base, 0).wait()
    for g in gathers(0):
        g.start()

    def pair_body(i, carry):
        c0 = base + 2 * i
        for b in range(2):
            c = c0 + b
            slot, other = b, 1 - b

            for g in gathers(slot):
                g.wait()

            @pl.when(c + 2 < end)
            def _idx_pf():
                idx_copy(c + 2, slot).start()

            @pl.when(c + 1 < end)
            def _gather_pf():
                idx_copy(c + 1, other).wait()
                for g in gathers(other):
                    g.start()

            @pl.when(c >= base + 2)
            def _out_drain():
                for o in out_copies(c - 2, slot):
                    o.wait()

            def tr_body(j, carry2):
                jh = jnp.full((16,), j // 128, jnp.int32)
                jv = jnp.full((16,), j % 128, jnp.int32)
                for k in range(CONTENT_DIM // 16):
                    x = rows_c[slot][j, pl.ds(k * 16, 16)]
                    plsc.store_scatter(qbuf[slot], [jh, didx_c[k], jv], x)
                x = rows_t[slot][j, pl.ds(0, 16)]
                plsc.store_scatter(qbuf[slot], [jh, didx_t, jv], x)
                return carry2

            lax.fori_loop(0, CHUNK, tr_body, 0, unroll=8)

            for o in out_copies(c, slot):
                o.start()
        return carry

    lax.fori_loop(0, CH_PER_W // 2, pair_body, 0)
    for o in out_copies(end - 2, 0):
        o.wait()
    for o in out_copies(end - 1, 1):
        o.wait()


@jax.jit
def kernel(token_ids, content_table, time_table):
    ids = token_ids.T.reshape(BATCH * SEQ // 128, 128)
    mesh = plsc.VectorSubcoreMesh(core_axis_name="c", subcore_axis_name="s")

    tail0 = NT_FULL * 128
    crm, trm = pl.kernel(
        _relayout_body,
        out_type=(
            jax.ShapeDtypeStruct((VOCAB // 2, 128), jnp.float32),
            jax.ShapeDtypeStruct((VOCAB // 8, 128), jnp.float32),
        ),
        mesh=mesh,
        scratch_types=[
            pltpu.VMEM((CONTENT_DIM, 128), jnp.float32),
            pltpu.VMEM((CONTENT_DIM, 128), jnp.float32),
            pltpu.VMEM((CONTENT_DIM, 128), jnp.float32),
            pltpu.VMEM((CONTENT_DIM, 128), jnp.float32),
            pltpu.VMEM((TIME_DIM, 128), jnp.float32),
            pltpu.VMEM((TIME_DIM, 128), jnp.float32),
            pltpu.VMEM((TIME_DIM, 128), jnp.float32),
            pltpu.VMEM((TIME_DIM, 128), jnp.float32),
        ] + [pltpu.SemaphoreType.DMA] * 8,
        compiler_params=pltpu.CompilerParams(
            use_tc_tiling_on_sc=True, needs_layout_passes=False),
    )(
        content_table.T,
        time_table.T,
        content_table[tail0:].reshape(32, 128),
        time_table[tail0:].reshape(8, 128),
    )
    content_rm = crm.reshape(VOCAB, CONTENT_DIM)
    time_rm = trm.reshape(VOCAB, TIME_DIM)

    q = pl.kernel(
        _gather_body,
        out_type=jax.ShapeDtypeStruct((SEQ, DT, 2 * BT, 8, 128), jnp.float32),
        mesh=mesh,
        scratch_types=[
            pltpu.VMEM((2, 128), jnp.int32),
            pltpu.VMEM((2, 128), jnp.int32),
            pltpu.VMEM((CHUNK, CONTENT_DIM), jnp.float32),
            pltpu.VMEM((CHUNK, CONTENT_DIM), jnp.float32),
            pltpu.VMEM((CHUNK, TIME_DIM), jnp.float32),
            pltpu.VMEM((CHUNK, TIME_DIM), jnp.float32),
            pltpu.VMEM((2, OUT_DIM, QPAD), jnp.float32),
            pltpu.VMEM((2, OUT_DIM, QPAD), jnp.float32),
        ] + [pltpu.SemaphoreType.DMA] * 8,
        compiler_params=pltpu.CompilerParams(
            use_tc_tiling_on_sc=False, needs_layout_passes=False),
    )(ids, content_rm, time_rm)
    return q.transpose(2, 4, 0, 1, 3).reshape(BATCH, SEQ, OUT_DIM)
